# Initial kernel scaffold; baseline (speedup 1.0000x reference)
import jax
import jax.numpy as jnp
from jax.experimental import pallas as pl


def kernel(input1, input2):
    raise NotImplementedError("write your pallas kernel here")


# speedup vs baseline: 50.7807x; 50.7807x over previous
import functools

import jax
import jax.numpy as jnp
from jax import lax
from jax.experimental import pallas as pl
from jax.experimental.pallas import tpu as pltpu
from jax.experimental.pallas import tpu_sc as plsc

H = 512
W = 512
HW = H * W
B = 8
NC = 2
NS = 16
L = 16
PX_PER_TILE = HW // NS
SEG = 4096
NSEG = PX_PER_TILE // SEG
CHUNK = 128
STEPS = CHUNK // L
NCHUNK = SEG // CHUNK
DRAIN = 2048
NPIECE = PX_PER_TILE // DRAIN


def _splat_body(fx_hbm, fy_hbm, dd_hbm, zeros_hbm, s3_hbm,
                fx_v, fy_v, dd_v, idx_v, sidx_v, vx_c, vy_c, dm_c, zc_v,
                deint_v, acc0, acc1, acc2):
    c = lax.axis_index("c")
    s = lax.axis_index("s")
    iota = lax.iota(jnp.int32, L)
    tile_base = s * PX_PER_TILE
    accs = (acc0, acc1, acc2)

    pltpu.sync_copy(zeros_hbm, zc_v)

    def write_seq_idx(base):
        for st in range(STEPS):
            sidx_v[pl.ds(st * L, L)] = base + st * L + iota

    def zero_body(z, carry):
        write_seq_idx(tile_base + z * CHUNK)
        for a in accs:
            pltpu.sync_copy(zc_v, a.at[sidx_v])
        return carry

    lax.fori_loop(0, PX_PER_TILE // CHUNK, zero_body, 0)

    def batch_body(k, carry):
        b = 2 * k + c
        in_off = b * HW + tile_base
        plsc.subcore_barrier()

        def seg_body(seg, carry2):
            seg_base = tile_base + seg * SEG
            pltpu.sync_copy(fx_hbm.at[pl.ds(in_off + seg * SEG, SEG)], fx_v)
            pltpu.sync_copy(fy_hbm.at[pl.ds(in_off + seg * SEG, SEG)], fy_v)
            pltpu.sync_copy(dd_hbm.at[pl.ds(in_off + seg * SEG, SEG)], dd_v)

            def chunk_body(ch, carry3):
                for st in range(STEPS):
                    i = pl.multiple_of(ch * CHUNK + st * L, L)
                    fx = fx_v[pl.ds(i, L)]
                    fy = fy_v[pl.ds(i, L)]
                    dd = dd_v[pl.ds(i, L)]
                    p = jnp.full((L,), seg_base, jnp.int32) + i + iota
                    xv = (p & (W - 1)).astype(jnp.float32)
                    yv = (p >> 9).astype(jnp.float32)
                    x2 = xv + fx
                    y2 = yv + fy
                    inb = ((x2 >= 0.0) & (x2 <= W - 1.0)
                           & (y2 >= 0.0) & (y2 <= H - 1.0))
                    xc = jnp.minimum(jnp.maximum(x2, 0.0), W - 1.0)
                    yc = jnp.minimum(jnp.maximum(y2, 0.0), H - 1.0)
                    ix = xc.astype(jnp.int32)
                    iy = yc.astype(jnp.int32)
                    lin = iy * W + ix
                    dm = jnp.where(inb, dd, 0.0)
                    idx_v[pl.ds(st * L, L)] = lin
                    vx_c[pl.ds(st * L, L)] = -fx * dm
                    vy_c[pl.ds(st * L, L)] = -fy * dm
                    dm_c[pl.ds(st * L, L)] = dm
                pltpu.sync_copy(vx_c, acc0.at[idx_v], add=True)
                pltpu.sync_copy(vy_c, acc1.at[idx_v], add=True)
                pltpu.sync_copy(dm_c, acc2.at[idx_v], add=True)
                return carry3

            lax.fori_loop(0, NCHUNK, chunk_body, 0)
            return carry2

        lax.fori_loop(0, NSEG, seg_body, 0)
        plsc.subcore_barrier()

        def piece_body(piece, carry2):
            row0 = tile_base + piece * DRAIN

            def drain_body(cp, carry3):
                write_seq_idx(row0 + cp * CHUNK)
                for chn, a in enumerate(accs):
                    pltpu.sync_copy(
                        a.at[sidx_v],
                        deint_v.at[pl.ds(chn * DRAIN + cp * CHUNK, CHUNK)])
                    pltpu.sync_copy(zc_v, a.at[sidx_v])
                return carry3

            lax.fori_loop(0, DRAIN // CHUNK, drain_body, 0)
            for chn in range(3):
                pltpu.sync_copy(deint_v.at[pl.ds(chn * DRAIN, DRAIN)],
                                s3_hbm.at[pl.ds((b * 3 + chn) * HW + row0,
                                                DRAIN)])
            return carry2

        lax.fori_loop(0, NPIECE, piece_body, 0)
        plsc.subcore_barrier()
        return carry

    lax.fori_loop(0, B // NC, batch_body, 0)


def _splat(fx, fy, dd, zeros):
    mesh = plsc.VectorSubcoreMesh(
        core_axis_name="c", subcore_axis_name="s",
        num_cores=NC, num_subcores=NS)
    f = pl.kernel(
        _splat_body,
        out_type=jax.ShapeDtypeStruct((B * 3 * HW,), jnp.float32),
        mesh=mesh,
        scratch_types=[
            pltpu.VMEM((SEG,), jnp.float32),
            pltpu.VMEM((SEG,), jnp.float32),
            pltpu.VMEM((SEG,), jnp.float32),
            pltpu.VMEM((CHUNK,), jnp.int32),
            pltpu.VMEM((CHUNK,), jnp.int32),
            pltpu.VMEM((CHUNK,), jnp.float32),
            pltpu.VMEM((CHUNK,), jnp.float32),
            pltpu.VMEM((CHUNK,), jnp.float32),
            pltpu.VMEM((CHUNK,), jnp.float32),
            pltpu.VMEM((3 * DRAIN,), jnp.float32),
            pltpu.VMEM_SHARED((HW,), jnp.float32),
            pltpu.VMEM_SHARED((HW,), jnp.float32),
            pltpu.VMEM_SHARED((HW,), jnp.float32),
        ],
        compiler_params=pltpu.CompilerParams(
            needs_layout_passes=False, use_tc_tiling_on_sc=False),
    )
    return f(fx, fy, dd, zeros)


def _shift(x, k, axis, reverse, pad):
    if axis == 1:
        blk = jnp.full((H, k), pad, x.dtype)
        if not reverse:
            return jnp.concatenate([blk, x[:, :W - k]], axis=1)
        return jnp.concatenate([x[:, k:], blk], axis=1)
    blk = jnp.full((k, W), pad, x.dtype)
    if not reverse:
        return jnp.concatenate([blk, x[:H - k, :]], axis=0)
    return jnp.concatenate([x[k:, :], blk], axis=0)


def _post_body(s3_ref, out_ref):
    lanes = lax.broadcasted_iota(jnp.int32, (H, W), 1)
    rows = lax.broadcasted_iota(jnp.int32, (H, W), 0)

    def xcomb(a):
        t = a + _shift(a, 1, 1, False, 0.0)
        return t + jnp.where(lanes == W - 1, a, 0.0)

    def ycomb(a):
        t = a + _shift(a, 1, 0, False, 0.0)
        return t + jnp.where(rows == H - 1, a, 0.0)

    uvx = ycomb(xcomb(s3_ref[0, 0]))
    uvy = ycomb(xcomb(s3_ref[0, 1]))
    ucnt = ycomb(xcomb(s3_ref[0, 2]))

    m = ucnt > 0.0
    safe = jnp.where(m, ucnt, 1.0)
    o0 = jnp.where(m, uvx / safe, 0.0)
    o1 = jnp.where(m, uvy / safe, 0.0)

    mf = jnp.where(m, 1.0, 0.0).astype(jnp.float32)

    def scan_dir(axis, reverse):
        v0, v1, mm = o0, o1, mf
        k = 1
        n = W if axis == 1 else H
        while k < n:
            v0s = _shift(v0, k, axis, reverse, 0.0)
            v1s = _shift(v1, k, axis, reverse, 0.0)
            mms = _shift(mm, k, axis, reverse, 0.0)
            keep = mm > 0.0
            v0 = jnp.where(keep, v0, v0s)
            v1 = jnp.where(keep, v1, v1s)
            mm = jnp.maximum(mm, mms)
            k *= 2
        return v0, v1, mm

    r0 = jnp.zeros((H, W), jnp.float32)
    r1 = jnp.zeros((H, W), jnp.float32)
    for axis, reverse in ((0, True), (0, False), (1, True), (1, False)):
        f0, f1, fm = scan_dir(axis, reverse)
        take = fm > 0.0
        r0 = jnp.where(take, f0, r0)
        r1 = jnp.where(take, f1, r1)

    out_ref[0, 0] = r0
    out_ref[0, 1] = r1


def _post(s3, interpret=False):
    return pl.pallas_call(
        _post_body,
        grid=(B,),
        in_specs=[pl.BlockSpec((1, 3, H, W), lambda b: (b, 0, 0, 0))],
        out_specs=pl.BlockSpec((1, 2, H, W), lambda b: (b, 0, 0, 0)),
        out_shape=jax.ShapeDtypeStruct((B, 2, H, W), jnp.float32),
        interpret=interpret,
    )(s3)


@jax.jit
def kernel(input1, input2):
    fx = input1[:, 0].reshape(B * HW)
    fy = input1[:, 1].reshape(B * HW)
    dd = input2[:, 0].reshape(B * HW)
    zeros = jnp.zeros((CHUNK,), jnp.float32)
    s3 = _splat(fx, fy, dd, zeros)
    return _post(s3.reshape(B, 3, H, W))


# speedup vs baseline: 73.0227x; 1.4380x over previous
import functools

import jax
import jax.numpy as jnp
from jax import lax
from jax.experimental import pallas as pl
from jax.experimental.pallas import tpu as pltpu
from jax.experimental.pallas import tpu_sc as plsc

H = 512
W = 512
HW = H * W
B = 8
NC = 2
NS = 16
L = 16
PX_PER_TILE = HW // NS
SEG = 4096
NSEG = PX_PER_TILE // SEG
CHUNK = 512
STEPS = CHUNK // L
NCHUNK = SEG // CHUNK
DRAIN = 2048
NPIECE = PX_PER_TILE // DRAIN


def _splat_body(fx_hbm, fy_hbm, dd_hbm, zeros_hbm, s3_hbm,
                fx_v, fy_v, dd_v, idx_v, sidx_v, vx_c, vy_c, dm_c, zc_v,
                deint_v, acc0, acc1, acc2):
    c = lax.axis_index("c")
    s = lax.axis_index("s")
    iota = lax.iota(jnp.int32, L)
    tile_base = s * PX_PER_TILE
    accs = (acc0, acc1, acc2)

    pltpu.sync_copy(zeros_hbm, zc_v)

    def write_seq_idx(base):
        for st in range(STEPS):
            sidx_v[pl.ds(st * L, L)] = base + st * L + iota

    def zero_body(z, carry):
        write_seq_idx(tile_base + z * CHUNK)
        for a in accs:
            pltpu.sync_copy(zc_v, a.at[sidx_v])
        return carry

    lax.fori_loop(0, PX_PER_TILE // CHUNK, zero_body, 0)

    def batch_body(k, carry):
        b = 2 * k + c
        in_off = b * HW + tile_base
        plsc.subcore_barrier()

        def seg_body(seg, carry2):
            seg_base = tile_base + seg * SEG
            pltpu.sync_copy(fx_hbm.at[pl.ds(in_off + seg * SEG, SEG)], fx_v)
            pltpu.sync_copy(fy_hbm.at[pl.ds(in_off + seg * SEG, SEG)], fy_v)
            pltpu.sync_copy(dd_hbm.at[pl.ds(in_off + seg * SEG, SEG)], dd_v)

            def chunk_body(ch, carry3):
                for st in range(STEPS):
                    i = pl.multiple_of(ch * CHUNK + st * L, L)
                    fx = fx_v[pl.ds(i, L)]
                    fy = fy_v[pl.ds(i, L)]
                    dd = dd_v[pl.ds(i, L)]
                    p = jnp.full((L,), seg_base, jnp.int32) + i + iota
                    xv = (p & (W - 1)).astype(jnp.float32)
                    yv = (p >> 9).astype(jnp.float32)
                    x2 = xv + fx
                    y2 = yv + fy
                    inb = ((x2 >= 0.0) & (x2 <= W - 1.0)
                           & (y2 >= 0.0) & (y2 <= H - 1.0))
                    xc = jnp.minimum(jnp.maximum(x2, 0.0), W - 1.0)
                    yc = jnp.minimum(jnp.maximum(y2, 0.0), H - 1.0)
                    ix = xc.astype(jnp.int32)
                    iy = yc.astype(jnp.int32)
                    lin = iy * W + ix
                    dm = jnp.where(inb, dd, 0.0)
                    idx_v[pl.ds(st * L, L)] = lin
                    vx_c[pl.ds(st * L, L)] = -fx * dm
                    vy_c[pl.ds(st * L, L)] = -fy * dm
                    dm_c[pl.ds(st * L, L)] = dm
                pltpu.sync_copy(vx_c, acc0.at[idx_v], add=True)
                pltpu.sync_copy(vy_c, acc1.at[idx_v], add=True)
                pltpu.sync_copy(dm_c, acc2.at[idx_v], add=True)
                return carry3

            lax.fori_loop(0, NCHUNK, chunk_body, 0)
            return carry2

        lax.fori_loop(0, NSEG, seg_body, 0)
        plsc.subcore_barrier()

        def piece_body(piece, carry2):
            row0 = tile_base + piece * DRAIN

            def drain_body(cp, carry3):
                write_seq_idx(row0 + cp * CHUNK)
                for chn, a in enumerate(accs):
                    pltpu.sync_copy(
                        a.at[sidx_v],
                        deint_v.at[pl.ds(chn * DRAIN + cp * CHUNK, CHUNK)])
                    pltpu.sync_copy(zc_v, a.at[sidx_v])
                return carry3

            lax.fori_loop(0, DRAIN // CHUNK, drain_body, 0)
            for chn in range(3):
                pltpu.sync_copy(deint_v.at[pl.ds(chn * DRAIN, DRAIN)],
                                s3_hbm.at[pl.ds((b * 3 + chn) * HW + row0,
                                                DRAIN)])
            return carry2

        lax.fori_loop(0, NPIECE, piece_body, 0)
        plsc.subcore_barrier()
        return carry

    lax.fori_loop(0, B // NC, batch_body, 0)


def _splat(fx, fy, dd, zeros):
    mesh = plsc.VectorSubcoreMesh(
        core_axis_name="c", subcore_axis_name="s",
        num_cores=NC, num_subcores=NS)
    f = pl.kernel(
        _splat_body,
        out_type=jax.ShapeDtypeStruct((B * 3 * HW,), jnp.float32),
        mesh=mesh,
        scratch_types=[
            pltpu.VMEM((SEG,), jnp.float32),
            pltpu.VMEM((SEG,), jnp.float32),
            pltpu.VMEM((SEG,), jnp.float32),
            pltpu.VMEM((CHUNK,), jnp.int32),
            pltpu.VMEM((CHUNK,), jnp.int32),
            pltpu.VMEM((CHUNK,), jnp.float32),
            pltpu.VMEM((CHUNK,), jnp.float32),
            pltpu.VMEM((CHUNK,), jnp.float32),
            pltpu.VMEM((CHUNK,), jnp.float32),
            pltpu.VMEM((3 * DRAIN,), jnp.float32),
            pltpu.VMEM_SHARED((HW,), jnp.float32),
            pltpu.VMEM_SHARED((HW,), jnp.float32),
            pltpu.VMEM_SHARED((HW,), jnp.float32),
        ],
        compiler_params=pltpu.CompilerParams(
            needs_layout_passes=False, use_tc_tiling_on_sc=False),
    )
    return f(fx, fy, dd, zeros)


def _shift(x, k, axis, reverse, pad):
    if axis == 1:
        blk = jnp.full((H, k), pad, x.dtype)
        if not reverse:
            return jnp.concatenate([blk, x[:, :W - k]], axis=1)
        return jnp.concatenate([x[:, k:], blk], axis=1)
    blk = jnp.full((k, W), pad, x.dtype)
    if not reverse:
        return jnp.concatenate([blk, x[:H - k, :]], axis=0)
    return jnp.concatenate([x[k:, :], blk], axis=0)


def _post_body(s3_ref, out_ref):
    lanes = lax.broadcasted_iota(jnp.int32, (H, W), 1)
    rows = lax.broadcasted_iota(jnp.int32, (H, W), 0)

    def xcomb(a):
        t = a + _shift(a, 1, 1, False, 0.0)
        return t + jnp.where(lanes == W - 1, a, 0.0)

    def ycomb(a):
        t = a + _shift(a, 1, 0, False, 0.0)
        return t + jnp.where(rows == H - 1, a, 0.0)

    uvx = ycomb(xcomb(s3_ref[0, 0]))
    uvy = ycomb(xcomb(s3_ref[0, 1]))
    ucnt = ycomb(xcomb(s3_ref[0, 2]))

    m = ucnt > 0.0
    safe = jnp.where(m, ucnt, 1.0)
    o0 = jnp.where(m, uvx / safe, 0.0)
    o1 = jnp.where(m, uvy / safe, 0.0)

    mf = jnp.where(m, 1.0, 0.0).astype(jnp.float32)

    def scan_dir(axis, reverse):
        v0, v1, mm = o0, o1, mf
        k = 1
        n = W if axis == 1 else H
        while k < n:
            v0s = _shift(v0, k, axis, reverse, 0.0)
            v1s = _shift(v1, k, axis, reverse, 0.0)
            mms = _shift(mm, k, axis, reverse, 0.0)
            keep = mm > 0.0
            v0 = jnp.where(keep, v0, v0s)
            v1 = jnp.where(keep, v1, v1s)
            mm = jnp.maximum(mm, mms)
            k *= 2
        return v0, v1, mm

    r0 = jnp.zeros((H, W), jnp.float32)
    r1 = jnp.zeros((H, W), jnp.float32)
    for axis, reverse in ((0, True), (0, False), (1, True), (1, False)):
        f0, f1, fm = scan_dir(axis, reverse)
        take = fm > 0.0
        r0 = jnp.where(take, f0, r0)
        r1 = jnp.where(take, f1, r1)

    out_ref[0, 0] = r0
    out_ref[0, 1] = r1


def _post(s3, interpret=False):
    return pl.pallas_call(
        _post_body,
        grid=(B,),
        in_specs=[pl.BlockSpec((1, 3, H, W), lambda b: (b, 0, 0, 0))],
        out_specs=pl.BlockSpec((1, 2, H, W), lambda b: (b, 0, 0, 0)),
        out_shape=jax.ShapeDtypeStruct((B, 2, H, W), jnp.float32),
        interpret=interpret,
    )(s3)


@jax.jit
def kernel(input1, input2):
    fx = input1[:, 0].reshape(B * HW)
    fy = input1[:, 1].reshape(B * HW)
    dd = input2[:, 0].reshape(B * HW)
    zeros = jnp.zeros((CHUNK,), jnp.float32)
    s3 = _splat(fx, fy, dd, zeros)
    return _post(s3.reshape(B, 3, H, W))


# speedup vs baseline: 75.2343x; 1.0303x over previous
import functools

import jax
import jax.numpy as jnp
from jax import lax
from jax.experimental import pallas as pl
from jax.experimental.pallas import tpu as pltpu
from jax.experimental.pallas import tpu_sc as plsc

H = 512
W = 512
HW = H * W
B = 8
NC = 2
NS = 16
L = 16
PX_PER_TILE = HW // NS
SEG = 4096
NSEG = PX_PER_TILE // SEG
CHUNK = 1024
STEPS = CHUNK // L
NCHUNK = SEG // CHUNK
DRAIN = 2048
NPIECE = PX_PER_TILE // DRAIN


def _splat_body(fx_hbm, fy_hbm, dd_hbm, zeros_hbm, s3_hbm,
                fx_v, fy_v, dd_v, idx_v, sidx_v, vx_c, vy_c, dm_c, zc_v,
                deint_v, acc0, acc1, acc2):
    c = lax.axis_index("c")
    s = lax.axis_index("s")
    iota = lax.iota(jnp.int32, L)
    tile_base = s * PX_PER_TILE
    accs = (acc0, acc1, acc2)

    pltpu.sync_copy(zeros_hbm, zc_v)

    def write_seq_idx(base):
        for st in range(STEPS):
            sidx_v[pl.ds(st * L, L)] = base + st * L + iota

    def zero_body(z, carry):
        write_seq_idx(tile_base + z * CHUNK)
        for a in accs:
            pltpu.sync_copy(zc_v, a.at[sidx_v])
        return carry

    lax.fori_loop(0, PX_PER_TILE // CHUNK, zero_body, 0)

    def batch_body(k, carry):
        b = 2 * k + c
        in_off = b * HW + tile_base
        plsc.subcore_barrier()

        def seg_body(seg, carry2):
            seg_base = tile_base + seg * SEG
            pltpu.sync_copy(fx_hbm.at[pl.ds(in_off + seg * SEG, SEG)], fx_v)
            pltpu.sync_copy(fy_hbm.at[pl.ds(in_off + seg * SEG, SEG)], fy_v)
            pltpu.sync_copy(dd_hbm.at[pl.ds(in_off + seg * SEG, SEG)], dd_v)

            def chunk_body(ch, carry3):
                for st in range(STEPS):
                    i = pl.multiple_of(ch * CHUNK + st * L, L)
                    fx = fx_v[pl.ds(i, L)]
                    fy = fy_v[pl.ds(i, L)]
                    dd = dd_v[pl.ds(i, L)]
                    p = jnp.full((L,), seg_base, jnp.int32) + i + iota
                    xv = (p & (W - 1)).astype(jnp.float32)
                    yv = (p >> 9).astype(jnp.float32)
                    x2 = xv + fx
                    y2 = yv + fy
                    inb = ((x2 >= 0.0) & (x2 <= W - 1.0)
                           & (y2 >= 0.0) & (y2 <= H - 1.0))
                    xc = jnp.minimum(jnp.maximum(x2, 0.0), W - 1.0)
                    yc = jnp.minimum(jnp.maximum(y2, 0.0), H - 1.0)
                    ix = xc.astype(jnp.int32)
                    iy = yc.astype(jnp.int32)
                    lin = iy * W + ix
                    dm = jnp.where(inb, dd, 0.0)
                    idx_v[pl.ds(st * L, L)] = lin
                    vx_c[pl.ds(st * L, L)] = -fx * dm
                    vy_c[pl.ds(st * L, L)] = -fy * dm
                    dm_c[pl.ds(st * L, L)] = dm
                pltpu.sync_copy(vx_c, acc0.at[idx_v], add=True)
                pltpu.sync_copy(vy_c, acc1.at[idx_v], add=True)
                pltpu.sync_copy(dm_c, acc2.at[idx_v], add=True)
                return carry3

            lax.fori_loop(0, NCHUNK, chunk_body, 0)
            return carry2

        lax.fori_loop(0, NSEG, seg_body, 0)
        plsc.subcore_barrier()

        def piece_body(piece, carry2):
            row0 = tile_base + piece * DRAIN

            def drain_body(cp, carry3):
                write_seq_idx(row0 + cp * CHUNK)
                for chn, a in enumerate(accs):
                    pltpu.sync_copy(
                        a.at[sidx_v],
                        deint_v.at[pl.ds(chn * DRAIN + cp * CHUNK, CHUNK)])
                    pltpu.sync_copy(zc_v, a.at[sidx_v])
                return carry3

            lax.fori_loop(0, DRAIN // CHUNK, drain_body, 0)
            for chn in range(3):
                pltpu.sync_copy(deint_v.at[pl.ds(chn * DRAIN, DRAIN)],
                                s3_hbm.at[pl.ds((b * 3 + chn) * HW + row0,
                                                DRAIN)])
            return carry2

        lax.fori_loop(0, NPIECE, piece_body, 0)
        plsc.subcore_barrier()
        return carry

    lax.fori_loop(0, B // NC, batch_body, 0)


def _splat(fx, fy, dd, zeros):
    mesh = plsc.VectorSubcoreMesh(
        core_axis_name="c", subcore_axis_name="s",
        num_cores=NC, num_subcores=NS)
    f = pl.kernel(
        _splat_body,
        out_type=jax.ShapeDtypeStruct((B * 3 * HW,), jnp.float32),
        mesh=mesh,
        scratch_types=[
            pltpu.VMEM((SEG,), jnp.float32),
            pltpu.VMEM((SEG,), jnp.float32),
            pltpu.VMEM((SEG,), jnp.float32),
            pltpu.VMEM((CHUNK,), jnp.int32),
            pltpu.VMEM((CHUNK,), jnp.int32),
            pltpu.VMEM((CHUNK,), jnp.float32),
            pltpu.VMEM((CHUNK,), jnp.float32),
            pltpu.VMEM((CHUNK,), jnp.float32),
            pltpu.VMEM((CHUNK,), jnp.float32),
            pltpu.VMEM((3 * DRAIN,), jnp.float32),
            pltpu.VMEM_SHARED((HW,), jnp.float32),
            pltpu.VMEM_SHARED((HW,), jnp.float32),
            pltpu.VMEM_SHARED((HW,), jnp.float32),
        ],
        compiler_params=pltpu.CompilerParams(
            needs_layout_passes=False, use_tc_tiling_on_sc=False),
    )
    return f(fx, fy, dd, zeros)


def _shift(x, k, axis, reverse, pad):
    if axis == 1:
        blk = jnp.full((H, k), pad, x.dtype)
        if not reverse:
            return jnp.concatenate([blk, x[:, :W - k]], axis=1)
        return jnp.concatenate([x[:, k:], blk], axis=1)
    blk = jnp.full((k, W), pad, x.dtype)
    if not reverse:
        return jnp.concatenate([blk, x[:H - k, :]], axis=0)
    return jnp.concatenate([x[k:, :], blk], axis=0)


def _post_body(s3_ref, out_ref):
    lanes = lax.broadcasted_iota(jnp.int32, (H, W), 1)
    rows = lax.broadcasted_iota(jnp.int32, (H, W), 0)

    def xcomb(a):
        t = a + _shift(a, 1, 1, False, 0.0)
        return t + jnp.where(lanes == W - 1, a, 0.0)

    def ycomb(a):
        t = a + _shift(a, 1, 0, False, 0.0)
        return t + jnp.where(rows == H - 1, a, 0.0)

    uvx = ycomb(xcomb(s3_ref[0, 0]))
    uvy = ycomb(xcomb(s3_ref[0, 1]))
    ucnt = ycomb(xcomb(s3_ref[0, 2]))

    m = ucnt > 0.0
    safe = jnp.where(m, ucnt, 1.0)
    o0 = jnp.where(m, uvx / safe, 0.0)
    o1 = jnp.where(m, uvy / safe, 0.0)

    mf = jnp.where(m, 1.0, 0.0).astype(jnp.float32)

    def scan_dir(axis, reverse):
        v0, v1, mm = o0, o1, mf
        k = 1
        n = W if axis == 1 else H
        while k < n:
            v0s = _shift(v0, k, axis, reverse, 0.0)
            v1s = _shift(v1, k, axis, reverse, 0.0)
            mms = _shift(mm, k, axis, reverse, 0.0)
            keep = mm > 0.0
            v0 = jnp.where(keep, v0, v0s)
            v1 = jnp.where(keep, v1, v1s)
            mm = jnp.maximum(mm, mms)
            k *= 2
        return v0, v1, mm

    r0 = jnp.zeros((H, W), jnp.float32)
    r1 = jnp.zeros((H, W), jnp.float32)
    for axis, reverse in ((0, True), (0, False), (1, True), (1, False)):
        f0, f1, fm = scan_dir(axis, reverse)
        take = fm > 0.0
        r0 = jnp.where(take, f0, r0)
        r1 = jnp.where(take, f1, r1)

    out_ref[0, 0] = r0
    out_ref[0, 1] = r1


def _post(s3, interpret=False):
    return pl.pallas_call(
        _post_body,
        grid=(B,),
        in_specs=[pl.BlockSpec((1, 3, H, W), lambda b: (b, 0, 0, 0))],
        out_specs=pl.BlockSpec((1, 2, H, W), lambda b: (b, 0, 0, 0)),
        out_shape=jax.ShapeDtypeStruct((B, 2, H, W), jnp.float32),
        interpret=interpret,
    )(s3)


@jax.jit
def kernel(input1, input2):
    fx = input1[:, 0].reshape(B * HW)
    fy = input1[:, 1].reshape(B * HW)
    dd = input2[:, 0].reshape(B * HW)
    zeros = jnp.zeros((CHUNK,), jnp.float32)
    s3 = _splat(fx, fy, dd, zeros)
    return _post(s3.reshape(B, 3, H, W))


# speedup vs baseline: 90.9267x; 1.2086x over previous
import functools

import jax
import jax.numpy as jnp
from jax import lax
from jax.experimental import pallas as pl
from jax.experimental.pallas import tpu as pltpu
from jax.experimental.pallas import tpu_sc as plsc

H = 512
W = 512
HW = H * W
B = 8
NC = 2
NS = 16
L = 16
PX_PER_TILE = HW // NS
SEG = 4096
NSEG = PX_PER_TILE // SEG
CHUNK = 1024
STEPS = CHUNK // L
NCHUNK = SEG // CHUNK
DRAIN = 2048
NPIECE = PX_PER_TILE // DRAIN


def _splat_body(fx_hbm, fy_hbm, dd_hbm, zeros_hbm, s3_hbm,
                fx_v, fy_v, dd_v, idx_v, sidx_v, vx_c, vy_c, dm_c, zc_v,
                deint_v, acc0, acc1, acc2, sem0, sem1, sem2):
    c = lax.axis_index("c")
    s = lax.axis_index("s")
    iota = lax.iota(jnp.int32, L)
    tile_base = s * PX_PER_TILE
    accs = (acc0, acc1, acc2)

    pltpu.sync_copy(zeros_hbm, zc_v)

    def write_seq_idx(base):
        for st in range(STEPS):
            sidx_v[pl.ds(st * L, L)] = base + st * L + iota

    sems = (sem0, sem1, sem2)

    def zero_body(z, carry):
        write_seq_idx(tile_base + z * CHUNK)
        ds = [pltpu.async_copy(zc_v, a.at[sidx_v], sm)
              for a, sm in zip(accs, sems)]
        for d in ds:
            d.wait()
        return carry

    lax.fori_loop(0, PX_PER_TILE // CHUNK, zero_body, 0)

    def batch_body(k, carry):
        b = 2 * k + c
        in_off = b * HW + tile_base
        plsc.subcore_barrier()

        def seg_body(seg, carry2):
            seg_base = tile_base + seg * SEG
            ds = [pltpu.async_copy(h.at[pl.ds(in_off + seg * SEG, SEG)], v, sm)
                  for h, v, sm in zip((fx_hbm, fy_hbm, dd_hbm),
                                      (fx_v, fy_v, dd_v), sems)]
            for d in ds:
                d.wait()

            def chunk_body(ch, carry3):
                for st in range(STEPS):
                    i = pl.multiple_of(ch * CHUNK + st * L, L)
                    fx = fx_v[pl.ds(i, L)]
                    fy = fy_v[pl.ds(i, L)]
                    dd = dd_v[pl.ds(i, L)]
                    p = jnp.full((L,), seg_base, jnp.int32) + i + iota
                    xv = (p & (W - 1)).astype(jnp.float32)
                    yv = (p >> 9).astype(jnp.float32)
                    x2 = xv + fx
                    y2 = yv + fy
                    inb = ((x2 >= 0.0) & (x2 <= W - 1.0)
                           & (y2 >= 0.0) & (y2 <= H - 1.0))
                    xc = jnp.minimum(jnp.maximum(x2, 0.0), W - 1.0)
                    yc = jnp.minimum(jnp.maximum(y2, 0.0), H - 1.0)
                    ix = xc.astype(jnp.int32)
                    iy = yc.astype(jnp.int32)
                    lin = iy * W + ix
                    dm = jnp.where(inb, dd, 0.0)
                    idx_v[pl.ds(st * L, L)] = lin
                    vx_c[pl.ds(st * L, L)] = -fx * dm
                    vy_c[pl.ds(st * L, L)] = -fy * dm
                    dm_c[pl.ds(st * L, L)] = dm
                ds = [pltpu.async_copy(v, a.at[idx_v], sm, add=True)
                      for v, a, sm in zip((vx_c, vy_c, dm_c), accs, sems)]
                for d in ds:
                    d.wait()
                return carry3

            lax.fori_loop(0, NCHUNK, chunk_body, 0)
            return carry2

        lax.fori_loop(0, NSEG, seg_body, 0)
        plsc.subcore_barrier()

        def piece_body(piece, carry2):
            row0 = tile_base + piece * DRAIN

            def drain_body(cp, carry3):
                write_seq_idx(row0 + cp * CHUNK)
                ds = [pltpu.async_copy(
                          a.at[sidx_v],
                          deint_v.at[pl.ds(chn * DRAIN + cp * CHUNK, CHUNK)],
                          sems[chn])
                      for chn, a in enumerate(accs)]
                for d in ds:
                    d.wait()
                ds = [pltpu.async_copy(zc_v, a.at[sidx_v], sm)
                      for a, sm in zip(accs, sems)]
                for d in ds:
                    d.wait()
                return carry3

            lax.fori_loop(0, DRAIN // CHUNK, drain_body, 0)
            ds = [pltpu.async_copy(
                      deint_v.at[pl.ds(chn * DRAIN, DRAIN)],
                      s3_hbm.at[pl.ds((b * 3 + chn) * HW + row0, DRAIN)],
                      sems[chn])
                  for chn in range(3)]
            for d in ds:
                d.wait()
            return carry2

        lax.fori_loop(0, NPIECE, piece_body, 0)
        plsc.subcore_barrier()
        return carry

    lax.fori_loop(0, B // NC, batch_body, 0)


def _splat(fx, fy, dd, zeros):
    mesh = plsc.VectorSubcoreMesh(
        core_axis_name="c", subcore_axis_name="s",
        num_cores=NC, num_subcores=NS)
    f = pl.kernel(
        _splat_body,
        out_type=jax.ShapeDtypeStruct((B * 3 * HW,), jnp.float32),
        mesh=mesh,
        scratch_types=[
            pltpu.VMEM((SEG,), jnp.float32),
            pltpu.VMEM((SEG,), jnp.float32),
            pltpu.VMEM((SEG,), jnp.float32),
            pltpu.VMEM((CHUNK,), jnp.int32),
            pltpu.VMEM((CHUNK,), jnp.int32),
            pltpu.VMEM((CHUNK,), jnp.float32),
            pltpu.VMEM((CHUNK,), jnp.float32),
            pltpu.VMEM((CHUNK,), jnp.float32),
            pltpu.VMEM((CHUNK,), jnp.float32),
            pltpu.VMEM((3 * DRAIN,), jnp.float32),
            pltpu.VMEM_SHARED((HW,), jnp.float32),
            pltpu.VMEM_SHARED((HW,), jnp.float32),
            pltpu.VMEM_SHARED((HW,), jnp.float32),
            pltpu.SemaphoreType.DMA,
            pltpu.SemaphoreType.DMA,
            pltpu.SemaphoreType.DMA,
        ],
        compiler_params=pltpu.CompilerParams(
            needs_layout_passes=False, use_tc_tiling_on_sc=False),
    )
    return f(fx, fy, dd, zeros)


def _shift(x, k, axis, reverse, pad):
    if axis == 1:
        blk = jnp.full((H, k), pad, x.dtype)
        if not reverse:
            return jnp.concatenate([blk, x[:, :W - k]], axis=1)
        return jnp.concatenate([x[:, k:], blk], axis=1)
    blk = jnp.full((k, W), pad, x.dtype)
    if not reverse:
        return jnp.concatenate([blk, x[:H - k, :]], axis=0)
    return jnp.concatenate([x[k:, :], blk], axis=0)


def _post_body(s3_ref, out_ref):
    lanes = lax.broadcasted_iota(jnp.int32, (H, W), 1)
    rows = lax.broadcasted_iota(jnp.int32, (H, W), 0)

    def xcomb(a):
        t = a + _shift(a, 1, 1, False, 0.0)
        return t + jnp.where(lanes == W - 1, a, 0.0)

    def ycomb(a):
        t = a + _shift(a, 1, 0, False, 0.0)
        return t + jnp.where(rows == H - 1, a, 0.0)

    uvx = ycomb(xcomb(s3_ref[0, 0]))
    uvy = ycomb(xcomb(s3_ref[0, 1]))
    ucnt = ycomb(xcomb(s3_ref[0, 2]))

    m = ucnt > 0.0
    safe = jnp.where(m, ucnt, 1.0)
    o0 = jnp.where(m, uvx / safe, 0.0)
    o1 = jnp.where(m, uvy / safe, 0.0)

    mf = jnp.where(m, 1.0, 0.0).astype(jnp.float32)

    def scan_dir(axis, reverse):
        v0, v1, mm = o0, o1, mf
        k = 1
        n = W if axis == 1 else H
        while k < n:
            v0s = _shift(v0, k, axis, reverse, 0.0)
            v1s = _shift(v1, k, axis, reverse, 0.0)
            mms = _shift(mm, k, axis, reverse, 0.0)
            keep = mm > 0.0
            v0 = jnp.where(keep, v0, v0s)
            v1 = jnp.where(keep, v1, v1s)
            mm = jnp.maximum(mm, mms)
            k *= 2
        return v0, v1, mm

    r0 = jnp.zeros((H, W), jnp.float32)
    r1 = jnp.zeros((H, W), jnp.float32)
    for axis, reverse in ((0, True), (0, False), (1, True), (1, False)):
        f0, f1, fm = scan_dir(axis, reverse)
        take = fm > 0.0
        r0 = jnp.where(take, f0, r0)
        r1 = jnp.where(take, f1, r1)

    out_ref[0, 0] = r0
    out_ref[0, 1] = r1


def _post(s3, interpret=False):
    return pl.pallas_call(
        _post_body,
        grid=(B,),
        in_specs=[pl.BlockSpec((1, 3, H, W), lambda b: (b, 0, 0, 0))],
        out_specs=pl.BlockSpec((1, 2, H, W), lambda b: (b, 0, 0, 0)),
        out_shape=jax.ShapeDtypeStruct((B, 2, H, W), jnp.float32),
        interpret=interpret,
    )(s3)


@jax.jit
def kernel(input1, input2):
    fx = input1[:, 0].reshape(B * HW)
    fy = input1[:, 1].reshape(B * HW)
    dd = input2[:, 0].reshape(B * HW)
    zeros = jnp.zeros((CHUNK,), jnp.float32)
    s3 = _splat(fx, fy, dd, zeros)
    return _post(s3.reshape(B, 3, H, W))
